# baseline (speedup 1.0000x reference)
import jax
import jax.numpy as jnp
import numpy as np
from jax import lax
from jax.experimental import pallas as pl
from jax.experimental.pallas import tpu as pltpu
from jax.experimental.pallas import tpu_sc as plsc

CALIBRATION_WEIGHT = 1.0

_N = 8388608
_NUM_BINS = 10
_BINS_PAD = 16
_NC, _NS, _L = 2, 16, 16
_NW = _NC * _NS
_PER_W = _N // _NW
_CHUNK = 8192
_VECS = _CHUNK // _L
_NCHUNKS = _PER_W // _CHUNK
_UNROLL = 8
_NBUF = 4

_KA = float(np.float32(10.0 * (1 - 2.0**-23)))
_BAD = float(np.uint32(0x3F666667).view(np.float32))


def _sc_partials_kernel(conf_hbm, acc_hbm, sumc_out, pack_out,
                        cbuf0, cbuf1, cbuf2, cbuf3, abuf0, abuf1, abuf2, abuf3,
                        sumc_ref, pack_ref, sem0, sem1, sem2, sem3):
    cbufs = [cbuf0, cbuf1, cbuf2, cbuf3]
    abufs = [abuf0, abuf1, abuf2, abuf3]
    wid = lax.axis_index("s") * _NC + lax.axis_index("c")
    base = pl.multiple_of(wid * _PER_W, 8)

    for b in range(_BINS_PAD):
        sumc_ref[b, :] = jnp.zeros((_L,), jnp.float32)
        pack_ref[b, :] = jnp.zeros((_L,), jnp.int32)

    lanes = lax.iota(jnp.int32, _L)
    sems = [sem0, sem1, sem2, sem3]

    def start(g):
        off = base + g * _CHUNK
        s = sems[g % _NBUF]
        pltpu.make_async_copy(conf_hbm.at[pl.ds(off, _CHUNK)], cbufs[g % _NBUF], s).start()
        pltpu.make_async_copy(acc_hbm.at[pl.ds(off, _CHUNK)], abufs[g % _NBUF], s).start()

    def wait(g):
        off = base + g * _CHUNK
        s = sems[g % _NBUF]
        pltpu.make_async_copy(conf_hbm.at[pl.ds(off, _CHUNK)], cbufs[g % _NBUF], s).wait()
        pltpu.make_async_copy(acc_hbm.at[pl.ds(off, _CHUNK)], abufs[g % _NBUF], s).wait()

    for g in range(_NBUF - 1):
        start(g)
    for g in range(_NCHUNKS):
        if g + _NBUF - 1 < _NCHUNKS:
            start(g + _NBUF - 1)
        wait(g)
        cb = cbufs[g % _NBUF]
        ab = abufs[g % _NBUF]

        @plsc.parallel_loop(0, _VECS, 1, unroll=_UNROLL)
        def body(i):
            o = i * _L
            c = cb[pl.ds(o, _L)]
            a = ab[pl.ds(o, _L)]
            ti = (c * _KA).astype(jnp.int32)
            idx = ti + jnp.where(c == _BAD, 1, 0)
            valid = c > 0.0
            x = a + 65536
            plsc.addupdate_scatter(sumc_ref, [idx, lanes], c, mask=valid)
            plsc.addupdate_scatter(pack_ref, [idx, lanes], x, mask=valid)

    pltpu.sync_copy(sumc_ref, sumc_out.at[wid])
    pltpu.sync_copy(pack_ref, pack_out.at[wid])


_sc_partials = pl.kernel(
    _sc_partials_kernel,
    out_type=(
        jax.ShapeDtypeStruct((_NW, _BINS_PAD, _L), jnp.float32),
        jax.ShapeDtypeStruct((_NW, _BINS_PAD, _L), jnp.int32),
    ),
    mesh=plsc.VectorSubcoreMesh(core_axis_name="c", subcore_axis_name="s"),
    scratch_types=[
        pltpu.VMEM((_CHUNK,), jnp.float32),
        pltpu.VMEM((_CHUNK,), jnp.float32),
        pltpu.VMEM((_CHUNK,), jnp.float32),
        pltpu.VMEM((_CHUNK,), jnp.float32),
        pltpu.VMEM((_CHUNK,), jnp.int32),
        pltpu.VMEM((_CHUNK,), jnp.int32),
        pltpu.VMEM((_CHUNK,), jnp.int32),
        pltpu.VMEM((_CHUNK,), jnp.int32),
        pltpu.VMEM((_BINS_PAD, _L), jnp.float32),
        pltpu.VMEM((_BINS_PAD, _L), jnp.int32),
        pltpu.SemaphoreType.DMA,
        pltpu.SemaphoreType.DMA,
        pltpu.SemaphoreType.DMA,
        pltpu.SemaphoreType.DMA,
    ],
    compiler_params=pltpu.CompilerParams(needs_layout_passes=False),
)


def _finish_body(pf_ref, pi_ref, o_ref):
    pf = pf_ref[...]
    pi = pi_ref[...]
    sumc = jnp.sum(pf, axis=(0, 2))
    cnt = jnp.sum(pi >> 16, axis=(0, 2)).astype(jnp.float32)
    suma = jnp.sum(pi & 65535, axis=(0, 2)).astype(jnp.float32)
    safe = jnp.maximum(cnt, 1.0)
    err = jnp.where(cnt > 0.0, (sumc / safe - suma / safe) ** 2, 0.0)
    o_ref[...] = jnp.reshape(jnp.sum(err), (1, 1))


_finish = pl.pallas_call(
    _finish_body,
    out_shape=jax.ShapeDtypeStruct((1, 1), jnp.float32),
)


def kernel(predicted_confidence, actual_accuracy, num_bins):
    sumc, packed = _sc_partials(predicted_confidence, actual_accuracy)
    total = _finish(sumc, packed)[0, 0]
    return CALIBRATION_WEIGHT * (total / num_bins)


# speedup vs baseline: 1.0362x; 1.0362x over previous
import jax
import jax.numpy as jnp
import numpy as np
from jax import lax
from jax.experimental import pallas as pl
from jax.experimental.pallas import tpu as pltpu
from jax.experimental.pallas import tpu_sc as plsc

CALIBRATION_WEIGHT = 1.0

_N = 8388608
_NUM_BINS = 10
_BINS_PAD = 16
_NC, _NS, _L = 2, 16, 16
_NW = _NC * _NS
_PER_W = _N // _NW
_CHUNK = 16384
_VECS = _CHUNK // _L
_NCHUNKS = _PER_W // _CHUNK
_UNROLL = 8
_NBUF = 3

_KA = float(np.float32(10.0 * (1 - 2.0**-23)))
_BAD = float(np.uint32(0x3F666667).view(np.float32))


def _sc_partials_kernel(conf_hbm, acc_hbm, sumc_out, pack_out,
                        cbuf0, cbuf1, cbuf2, abuf0, abuf1, abuf2,
                        sumc_ref, pack_ref, sem0, sem1, sem2):
    cbufs = [cbuf0, cbuf1, cbuf2]
    abufs = [abuf0, abuf1, abuf2]
    wid = lax.axis_index("s") * _NC + lax.axis_index("c")
    base = pl.multiple_of(wid * _PER_W, 8)

    for b in range(_BINS_PAD):
        sumc_ref[b, :] = jnp.zeros((_L,), jnp.float32)
        pack_ref[b, :] = jnp.zeros((_L,), jnp.int32)

    lanes = lax.iota(jnp.int32, _L)
    sems = [sem0, sem1, sem2]

    def start(g):
        off = base + g * _CHUNK
        s = sems[g % _NBUF]
        pltpu.make_async_copy(conf_hbm.at[pl.ds(off, _CHUNK)], cbufs[g % _NBUF], s).start()
        pltpu.make_async_copy(acc_hbm.at[pl.ds(off, _CHUNK)], abufs[g % _NBUF], s).start()

    def wait(g):
        off = base + g * _CHUNK
        s = sems[g % _NBUF]
        pltpu.make_async_copy(conf_hbm.at[pl.ds(off, _CHUNK)], cbufs[g % _NBUF], s).wait()
        pltpu.make_async_copy(acc_hbm.at[pl.ds(off, _CHUNK)], abufs[g % _NBUF], s).wait()

    for g in range(_NBUF - 1):
        start(g)
    for g in range(_NCHUNKS):
        if g + _NBUF - 1 < _NCHUNKS:
            start(g + _NBUF - 1)
        wait(g)
        cb = cbufs[g % _NBUF]
        ab = abufs[g % _NBUF]

        @plsc.parallel_loop(0, _VECS, 1, unroll=_UNROLL)
        def body(i):
            o = i * _L
            c = cb[pl.ds(o, _L)]
            a = ab[pl.ds(o, _L)]
            ti = (c * _KA).astype(jnp.int32)
            idx = ti + jnp.where(c == _BAD, 1, 0)
            valid = c > 0.0
            x = a + 65536
            plsc.addupdate_scatter(sumc_ref, [idx, lanes], c, mask=valid)
            plsc.addupdate_scatter(pack_ref, [idx, lanes], x, mask=valid)

    pltpu.sync_copy(sumc_ref, sumc_out.at[wid])
    pltpu.sync_copy(pack_ref, pack_out.at[wid])


_sc_partials = pl.kernel(
    _sc_partials_kernel,
    out_type=(
        jax.ShapeDtypeStruct((_NW, _BINS_PAD, _L), jnp.float32),
        jax.ShapeDtypeStruct((_NW, _BINS_PAD, _L), jnp.int32),
    ),
    mesh=plsc.VectorSubcoreMesh(core_axis_name="c", subcore_axis_name="s"),
    scratch_types=[
        pltpu.VMEM((_CHUNK,), jnp.float32),
        pltpu.VMEM((_CHUNK,), jnp.float32),
        pltpu.VMEM((_CHUNK,), jnp.float32),
        pltpu.VMEM((_CHUNK,), jnp.int32),
        pltpu.VMEM((_CHUNK,), jnp.int32),
        pltpu.VMEM((_CHUNK,), jnp.int32),
        pltpu.VMEM((_BINS_PAD, _L), jnp.float32),
        pltpu.VMEM((_BINS_PAD, _L), jnp.int32),
        pltpu.SemaphoreType.DMA,
        pltpu.SemaphoreType.DMA,
        pltpu.SemaphoreType.DMA,
    ],
    compiler_params=pltpu.CompilerParams(needs_layout_passes=False),
)


def _finish_body(pf_ref, pi_ref, o_ref):
    pf = pf_ref[...]
    pi = pi_ref[...]
    sumc = jnp.sum(pf, axis=(0, 2))
    cnt = jnp.sum(pi >> 16, axis=(0, 2)).astype(jnp.float32)
    suma = jnp.sum(pi & 65535, axis=(0, 2)).astype(jnp.float32)
    safe = jnp.maximum(cnt, 1.0)
    err = jnp.where(cnt > 0.0, (sumc / safe - suma / safe) ** 2, 0.0)
    o_ref[...] = jnp.reshape(jnp.sum(err), (1, 1))


_finish = pl.pallas_call(
    _finish_body,
    out_shape=jax.ShapeDtypeStruct((1, 1), jnp.float32),
)


def kernel(predicted_confidence, actual_accuracy, num_bins):
    sumc, packed = _sc_partials(predicted_confidence, actual_accuracy)
    total = _finish(sumc, packed)[0, 0]
    return CALIBRATION_WEIGHT * (total / num_bins)
